# initial kernel scaffold (unmeasured)
import jax
import jax.numpy as jnp
from jax import lax
from jax.experimental import pallas as pl
from jax.experimental.pallas import tpu as pltpu

N_DEV = 16


def _snap_e4m3(v):
    a = jnp.abs(v)
    u = lax.bitcast_convert_type(a, jnp.uint32)
    lsb = (u >> 20) & jnp.uint32(1)
    r = (u + jnp.uint32(0x7FFFF) + lsb) & jnp.uint32(0xFFF00000)
    normal = lax.bitcast_convert_type(r, jnp.float32)
    sub = jnp.round(a * 512.0) * (1.0 / 512.0)
    snapped = jnp.where(a >= 2.0**-6, normal, sub)
    snapped = jnp.minimum(snapped, 448.0)
    return jnp.sign(v) * snapped


def kernel(x, w_mat):
    k_glob, k_per = x.shape
    m_per = k_glob // N_DEV
    n = w_mat.shape[1]

    def body(x_ref, w_ref, out_ref, xrows_ref, amax_ref,
             send_sems, recv_sems, a_send_sems, a_recv_sems):
        my = lax.axis_index("i")

        xrows_ref[:, pl.ds(my * k_per, k_per)] = x_ref[pl.ds(my * m_per, m_per), :]

        sends = []
        for h in range(1, N_DEV):
            peer = lax.rem(my + h, N_DEV)
            rdma = pltpu.make_async_remote_copy(
                src_ref=x_ref.at[pl.ds(peer * m_per, m_per), :],
                dst_ref=xrows_ref.at[:, pl.ds(my * k_per, k_per)],
                send_sem=send_sems.at[peer],
                recv_sem=recv_sems.at[my],
                device_id=(peer,),
                device_id_type=pl.DeviceIdType.MESH,
            )
            rdma.start()
            sends.append(rdma)

        for h in range(1, N_DEV):
            j = lax.rem(my + h, N_DEV)
            recv = pltpu.make_async_remote_copy(
                src_ref=x_ref.at[pl.ds(0, m_per), :],
                dst_ref=xrows_ref.at[:, pl.ds(j * k_per, k_per)],
                send_sem=send_sems.at[my],
                recv_sem=recv_sems.at[j],
                device_id=(my,),
                device_id_type=pl.DeviceIdType.MESH,
            )
            recv.wait_recv()
        for rdma in sends:
            rdma.wait_send()

        y = jnp.dot(xrows_ref[:, :], w_ref[:, :],
                    preferred_element_type=jnp.float32)

        amax_local = jnp.max(jnp.abs(y))
        amax_ref[pl.ds(my, 1), :] = jnp.full((1, 128), amax_local, jnp.float32)

        a_sends = []
        for h in range(1, N_DEV):
            peer = lax.rem(my + h, N_DEV)
            rdma = pltpu.make_async_remote_copy(
                src_ref=amax_ref.at[pl.ds(my, 1), :],
                dst_ref=amax_ref.at[pl.ds(my, 1), :],
                send_sem=a_send_sems.at[peer],
                recv_sem=a_recv_sems.at[my],
                device_id=(peer,),
                device_id_type=pl.DeviceIdType.MESH,
            )
            rdma.start()
            a_sends.append(rdma)
        for h in range(1, N_DEV):
            j = lax.rem(my + h, N_DEV)
            recv = pltpu.make_async_remote_copy(
                src_ref=amax_ref.at[pl.ds(0, 1), :],
                dst_ref=amax_ref.at[pl.ds(j, 1), :],
                send_sem=a_send_sems.at[my],
                recv_sem=a_recv_sems.at[j],
                device_id=(my,),
                device_id_type=pl.DeviceIdType.MESH,
            )
            recv.wait_recv()
        for rdma in a_sends:
            rdma.wait_send()

        amax = jnp.max(amax_ref[:, :])

        scale = amax * (1.0 / 448.0)
        out_ref[:, :] = _snap_e4m3(y * pl.reciprocal(scale)) * scale

    return pl.pallas_call(
        body,
        out_shape=jax.ShapeDtypeStruct((m_per, n), jnp.float32),
        in_specs=[
            pl.BlockSpec(memory_space=pltpu.VMEM),
            pl.BlockSpec(memory_space=pltpu.VMEM),
        ],
        out_specs=pl.BlockSpec(memory_space=pltpu.VMEM),
        scratch_shapes=[
            pltpu.VMEM((m_per, N_DEV * k_per), jnp.float32),
            pltpu.VMEM((N_DEV, 128), jnp.float32),
            pltpu.SemaphoreType.DMA((N_DEV,)),
            pltpu.SemaphoreType.DMA((N_DEV,)),
            pltpu.SemaphoreType.DMA((N_DEV,)),
            pltpu.SemaphoreType.DMA((N_DEV,)),
        ],
        compiler_params=pltpu.CompilerParams(collective_id=0),
    )(x, w_mat)


# baseline (device time: 90125 ns/iter reference)
import jax
import jax.numpy as jnp
from jax import lax
from jax.experimental import pallas as pl
from jax.experimental.pallas import tpu as pltpu

N_DEV = 16


def _snap_e4m3(v):
    a = jnp.abs(v)
    u = lax.bitcast_convert_type(a, jnp.uint32)
    lsb = (u >> 20) & jnp.uint32(1)
    r = (u + jnp.uint32(0x7FFFF) + lsb) & jnp.uint32(0xFFF00000)
    normal = lax.bitcast_convert_type(r, jnp.float32)
    sub = jnp.round(a * 512.0) * (1.0 / 512.0)
    snapped = jnp.where(a >= 2.0**-6, normal, sub)
    snapped = jnp.minimum(snapped, 448.0)
    return jnp.sign(v) * snapped


def kernel(x, w_mat):
    k_glob, k_per = x.shape
    m_per = k_glob // N_DEV
    n = w_mat.shape[1]

    def body(x_ref, w_ref, out_ref, xrows_ref, amax_ref,
             send_sems, recv_sems, a_send_sems, a_recv_sems):
        my = lax.axis_index("i")

        xrows_ref[:, pl.ds(my * k_per, k_per)] = x_ref[pl.ds(my * m_per, m_per), :]

        sends = []
        for h in range(1, N_DEV):
            peer = lax.rem(my + h, N_DEV)
            rdma = pltpu.make_async_remote_copy(
                src_ref=x_ref.at[pl.ds(peer * m_per, m_per), :],
                dst_ref=xrows_ref.at[:, pl.ds(my * k_per, k_per)],
                send_sem=send_sems.at[peer],
                recv_sem=recv_sems.at[my],
                device_id=(peer,),
                device_id_type=pl.DeviceIdType.MESH,
            )
            rdma.start()
            sends.append(rdma)

        for h in range(1, N_DEV):
            j = lax.rem(my + h, N_DEV)
            recv = pltpu.make_async_remote_copy(
                src_ref=x_ref.at[pl.ds(0, m_per), :],
                dst_ref=xrows_ref.at[:, pl.ds(j * k_per, k_per)],
                send_sem=send_sems.at[my],
                recv_sem=recv_sems.at[j],
                device_id=(my,),
                device_id_type=pl.DeviceIdType.MESH,
            )
            recv.wait_recv()
        for rdma in sends:
            rdma.wait_send()

        y = jnp.dot(xrows_ref[:, :], w_ref[:, :],
                    preferred_element_type=jnp.float32)

        amax_local = jnp.max(jnp.abs(y))
        amax_ref[pl.ds(my, 1), :] = jnp.full((1, 128), amax_local, jnp.float32)

        a_sends = []
        for h in range(1, N_DEV):
            peer = lax.rem(my + h, N_DEV)
            rdma = pltpu.make_async_remote_copy(
                src_ref=amax_ref.at[pl.ds(my, 1), :],
                dst_ref=amax_ref.at[pl.ds(my, 1), :],
                send_sem=a_send_sems.at[peer],
                recv_sem=a_recv_sems.at[my],
                device_id=(peer,),
                device_id_type=pl.DeviceIdType.MESH,
            )
            rdma.start()
            a_sends.append(rdma)
        for h in range(1, N_DEV):
            j = lax.rem(my + h, N_DEV)
            recv = pltpu.make_async_remote_copy(
                src_ref=amax_ref.at[pl.ds(0, 1), :],
                dst_ref=amax_ref.at[pl.ds(j, 1), :],
                send_sem=a_send_sems.at[my],
                recv_sem=a_recv_sems.at[j],
                device_id=(my,),
                device_id_type=pl.DeviceIdType.MESH,
            )
            recv.wait_recv()
        for rdma in a_sends:
            rdma.wait_send()

        amax = jnp.max(amax_ref[:, :])

        scale = amax * (1.0 / 448.0)
        out_ref[:, :] = _snap_e4m3(y / scale) * scale

    return pl.pallas_call(
        body,
        out_shape=jax.ShapeDtypeStruct((m_per, n), jnp.float32),
        in_specs=[
            pl.BlockSpec(memory_space=pltpu.VMEM),
            pl.BlockSpec(memory_space=pltpu.VMEM),
        ],
        out_specs=pl.BlockSpec(memory_space=pltpu.VMEM),
        scratch_shapes=[
            pltpu.VMEM((m_per, N_DEV * k_per), jnp.float32),
            pltpu.VMEM((N_DEV, 128), jnp.float32),
            pltpu.SemaphoreType.DMA((N_DEV,)),
            pltpu.SemaphoreType.DMA((N_DEV,)),
            pltpu.SemaphoreType.DMA((N_DEV,)),
            pltpu.SemaphoreType.DMA((N_DEV,)),
        ],
        compiler_params=pltpu.CompilerParams(
            vmem_limit_bytes=60 * 1024 * 1024,
        ),
    )(x, w_mat)


# device time: 64293 ns/iter; 1.4018x vs baseline; 1.4018x over previous
import jax
import jax.numpy as jnp
from jax import lax
from jax.experimental import pallas as pl
from jax.experimental.pallas import tpu as pltpu

N_DEV = 16


def _snap_e4m3(v):
    a = jnp.abs(v)
    u = lax.bitcast_convert_type(a, jnp.uint32)
    lsb = (u >> 20) & jnp.uint32(1)
    r = (u + jnp.uint32(0x7FFFF) + lsb) & jnp.uint32(0xFFF00000)
    normal = lax.bitcast_convert_type(r, jnp.float32)
    sub = jnp.round(a * 512.0) * (1.0 / 512.0)
    snapped = jnp.where(a >= 2.0**-6, normal, sub)
    snapped = jnp.minimum(snapped, 448.0)
    return jnp.sign(v) * snapped


def kernel(x, w_mat):
    k_glob, k_per = x.shape
    m_per = k_glob // N_DEV
    n = w_mat.shape[1]

    def body(x_ref, w_ref, out_ref, xbf_ref, comm_ref, amax_ref,
             send_sems, recv_sems, a_send_sems, a_recv_sems):
        my = lax.axis_index("i")

        def block_rdma(peer):
            return pltpu.make_async_remote_copy(
                src_ref=xbf_ref.at[pl.ds(peer * m_per, m_per), :],
                dst_ref=comm_ref.at[my],
                send_sem=send_sems.at[peer],
                recv_sem=recv_sems.at[my],
                device_id=(peer,),
                device_id_type=pl.DeviceIdType.MESH,
            )

        def block_recv(j):
            return pltpu.make_async_remote_copy(
                src_ref=comm_ref.at[j],
                dst_ref=comm_ref.at[j],
                send_sem=send_sems.at[my],
                recv_sem=recv_sems.at[j],
                device_id=(my,),
                device_id_type=pl.DeviceIdType.MESH,
            )

        xbf_ref[:, :] = x_ref[:, :].astype(jnp.bfloat16)

        def send_body(h, c):
            block_rdma(lax.rem(my + h, N_DEV)).start()
            return c

        lax.fori_loop(1, N_DEV, send_body, 0)

        comm_ref[my] = xbf_ref[pl.ds(my * m_per, m_per), :]

        def chunk_gemm(j):
            xj = comm_ref[j]
            wj = w_ref[pl.ds(j * k_per, k_per), :].astype(jnp.bfloat16)
            return jnp.dot(xj, wj, preferred_element_type=jnp.float32)

        def recv_body(h, y):
            j = lax.rem(my + (N_DEV - h), N_DEV)
            block_recv(j).wait_recv()
            return y + chunk_gemm(j)

        y = lax.fori_loop(1, N_DEV, recv_body, chunk_gemm(my))

        def wait_send_body(h, c):
            block_rdma(lax.rem(my + h, N_DEV)).wait_send()
            return c

        lax.fori_loop(1, N_DEV, wait_send_body, 0)

        amax_local = jnp.max(jnp.abs(y))
        amax_ref[pl.ds(my, 1), :] = jnp.full((1, 128), amax_local, jnp.float32)

        def amax_rdma(peer):
            return pltpu.make_async_remote_copy(
                src_ref=amax_ref.at[pl.ds(my, 1), :],
                dst_ref=amax_ref.at[pl.ds(my, 1), :],
                send_sem=a_send_sems.at[peer],
                recv_sem=a_recv_sems.at[my],
                device_id=(peer,),
                device_id_type=pl.DeviceIdType.MESH,
            )

        def a_send_body(h, c):
            amax_rdma(lax.rem(my + h, N_DEV)).start()
            return c

        lax.fori_loop(1, N_DEV, a_send_body, 0)

        def a_recv_body(h, c):
            j = lax.rem(my + h, N_DEV)
            recv = pltpu.make_async_remote_copy(
                src_ref=amax_ref.at[pl.ds(0, 1), :],
                dst_ref=amax_ref.at[pl.ds(j, 1), :],
                send_sem=a_send_sems.at[my],
                recv_sem=a_recv_sems.at[j],
                device_id=(my,),
                device_id_type=pl.DeviceIdType.MESH,
            )
            recv.wait_recv()
            return c

        lax.fori_loop(1, N_DEV, a_recv_body, 0)

        def a_wait_send_body(h, c):
            amax_rdma(lax.rem(my + h, N_DEV)).wait_send()
            return c

        lax.fori_loop(1, N_DEV, a_wait_send_body, 0)

        amax = jnp.max(amax_ref[:, :])

        scale = amax * (1.0 / 448.0)
        out_ref[:, :] = _snap_e4m3(y / scale) * scale

    return pl.pallas_call(
        body,
        out_shape=jax.ShapeDtypeStruct((m_per, n), jnp.float32),
        in_specs=[
            pl.BlockSpec(memory_space=pltpu.VMEM),
            pl.BlockSpec(memory_space=pltpu.VMEM),
        ],
        out_specs=pl.BlockSpec(memory_space=pltpu.VMEM),
        scratch_shapes=[
            pltpu.VMEM((k_glob, k_per), jnp.bfloat16),
            pltpu.VMEM((N_DEV, m_per, k_per), jnp.bfloat16),
            pltpu.VMEM((N_DEV, 128), jnp.float32),
            pltpu.SemaphoreType.DMA((N_DEV,)),
            pltpu.SemaphoreType.DMA((N_DEV,)),
            pltpu.SemaphoreType.DMA((N_DEV,)),
            pltpu.SemaphoreType.DMA((N_DEV,)),
        ],
        compiler_params=pltpu.CompilerParams(
            vmem_limit_bytes=60 * 1024 * 1024,
        ),
    )(x, w_mat)
